# baseline (speedup 1.0000x reference)
import functools
import math

import jax
import jax.numpy as jnp
from jax import lax
from jax.experimental import pallas as pl
from jax.experimental.pallas import tpu as pltpu
from jax.experimental.pallas import tpu_sc as plsc

_N = 10000
_E = 320000
_D = 128
_D_OUT = 64
_ALPHA = 0.1
_THETA = 0.5

_NC = 2
_NS = 16
_NW = _NC * _NS
_G = 128
_NGRP = 80
_EPT = _G * _NGRP
_EPAD = _EPT * _NW
_NPAD = 10240
_RPT = _NPAD // _NS

_ROW_BLK = 1000
_NBUF = 2
_NIDX = 8
_LOOK = 4


def _sc_agg_body(h_hbm, src_hbm, dst_hbm, out_hbm,
                 isrc_v, idst_v, rows_v, agg_sh, gsem, ssem, isem):
    c = lax.axis_index("c")
    s = lax.axis_index("s")
    w = c * _NS + s

    def _zrow(i, _):
        for j in range(_D // 16):
            rows_v[0, i, pl.ds(j * 16, 16)] = jnp.zeros((16,), jnp.float32)
        return 0
    lax.fori_loop(0, _G, _zrow, 0)
    for k in range(_RPT // _G):
        pltpu.sync_copy(rows_v.at[0], agg_sh.at[pl.ds(s * _RPT + k * _G, _G)])
    plsc.subcore_barrier()

    def _fire_idx(g):
        slot = lax.rem(g, _NIDX)
        pltpu.async_copy(src_hbm.at[w, g], isrc_v.at[slot], isem)
        pltpu.async_copy(dst_hbm.at[w, g], idst_v.at[slot], isem)

    def _wait_idx(g):
        slot = lax.rem(g, _NIDX)
        pltpu.make_async_copy(src_hbm.at[w, g], isrc_v.at[slot], isem).wait()
        pltpu.make_async_copy(dst_hbm.at[w, g], idst_v.at[slot], isem).wait()

    def _fire_gather(g):
        pltpu.async_copy(h_hbm.at[isrc_v.at[lax.rem(g, _NIDX)]],
                         rows_v.at[lax.rem(g, _NBUF)], gsem)

    def _wait_gather(g):
        pltpu.make_async_copy(h_hbm.at[isrc_v.at[lax.rem(g, _NIDX)]],
                              rows_v.at[lax.rem(g, _NBUF)], gsem).wait()

    def _fire_scatter(g):
        pltpu.async_copy(rows_v.at[lax.rem(g, _NBUF)],
                         agg_sh.at[idst_v.at[lax.rem(g, _NIDX)]], ssem,
                         add=True)

    def _wait_scatter(g):
        pltpu.make_async_copy(rows_v.at[lax.rem(g, _NBUF)],
                              agg_sh.at[idst_v.at[lax.rem(g, _NIDX)]],
                              ssem).wait()

    for g in range(_LOOK + 2):
        _fire_idx(g)
    _wait_idx(0)
    _fire_gather(0)
    _wait_idx(1)
    _fire_gather(1)
    _wait_gather(0)
    _fire_scatter(0)

    def _edge_grp(g, _):
        _wait_gather(g)
        _wait_scatter(g - 1)
        _fire_scatter(g)
        _wait_idx(g + 1)
        _fire_gather(g + 1)
        _fire_idx(g + _LOOK + 1)
        return 0
    lax.fori_loop(1, _NGRP - _LOOK - 1, _edge_grp, 0)

    for g in range(_NGRP - _LOOK - 1, _NGRP - 1):
        _wait_gather(g)
        _wait_scatter(g - 1)
        _fire_scatter(g)
        _wait_idx(g + 1)
        _fire_gather(g + 1)
    g = _NGRP - 1
    _wait_gather(g)
    _wait_scatter(g - 1)
    _fire_scatter(g)
    _wait_scatter(g)
    plsc.subcore_barrier()

    pltpu.sync_copy(agg_sh.at[pl.ds(s * _RPT, _RPT)],
                    out_hbm.at[c, pl.ds(s * _RPT, _RPT)])


_sc_agg = pl.kernel(
    _sc_agg_body,
    out_type=jax.ShapeDtypeStruct((_NC, _NPAD, _D), jnp.float32),
    mesh=plsc.VectorSubcoreMesh(core_axis_name="c", subcore_axis_name="s"),
    scratch_types=[
        pltpu.VMEM((_NIDX, _G), jnp.int32),
        pltpu.VMEM((_NIDX, _G), jnp.int32),
        pltpu.VMEM((_NBUF, _G, _D), jnp.float32),
        pltpu.VMEM_SHARED((_NPAD, _D), jnp.float32),
        pltpu.SemaphoreType.DMA,
        pltpu.SemaphoreType.DMA,
        pltpu.SemaphoreType.DMA,
    ],
)


def _in_proj_body(x_ref, w_ref, b_ref, o_ref):
    o_ref[...] = jnp.maximum(
        jnp.dot(x_ref[...], w_ref[...], preferred_element_type=jnp.float32)
        + b_ref[...], 0.0)


def _mix_body(beta, agg_ref, x0_ref, w_ref, o_ref):
    m = (1.0 - _ALPHA) * (agg_ref[0] + agg_ref[1]) + _ALPHA * x0_ref[...]
    o_ref[...] = jnp.maximum(
        (1.0 - beta) * m
        + beta * jnp.dot(m, w_ref[...], preferred_element_type=jnp.float32),
        0.0)


def _tail_body(beta, agg_ref, x0_ref, w_ref, wo_ref, bo_ref, o_ref):
    m = (1.0 - _ALPHA) * (agg_ref[0] + agg_ref[1]) + _ALPHA * x0_ref[...]
    h2 = jnp.maximum(
        (1.0 - beta) * m
        + beta * jnp.dot(m, w_ref[...], preferred_element_type=jnp.float32),
        0.0)
    z = jnp.dot(h2, wo_ref[...], preferred_element_type=jnp.float32) + bo_ref[...]
    z = z - jnp.max(z, axis=-1, keepdims=True)
    o_ref[...] = z - jnp.log(jnp.sum(jnp.exp(z), axis=-1, keepdims=True))


def kernel(x, edge_index, W_in, b_in, W_out, b_out, W_conv1, W_conv2):
    src = edge_index[0].astype(jnp.int32)
    dst = edge_index[1].astype(jnp.int32)
    pad = _EPAD - _E
    pad_dst = _N + (jnp.arange(pad, dtype=jnp.int32) % (_NPAD - _N))
    pad_src = jnp.arange(pad, dtype=jnp.int32) % _N
    src_p = jnp.concatenate([src, pad_src]).reshape(_NW, _NGRP, _G)
    dst_p = jnp.concatenate([dst, pad_dst]).reshape(_NW, _NGRP, _G)

    w_spec = pl.BlockSpec((_D, _D), lambda i: (0, 0))
    b_spec = pl.BlockSpec((1, _D), lambda i: (0, 0))
    row_spec = pl.BlockSpec((_ROW_BLK, _D), lambda i: (i, 0))
    agg_spec = pl.BlockSpec((_NC, _ROW_BLK, _D), lambda i: (0, i, 0))
    wo_spec = pl.BlockSpec((_D, _D_OUT), lambda i: (0, 0))
    bo_spec = pl.BlockSpec((1, _D_OUT), lambda i: (0, 0))
    grid = (_N // _ROW_BLK,)

    h = pl.pallas_call(
        _in_proj_body,
        grid=grid,
        in_specs=[row_spec, w_spec, b_spec],
        out_specs=row_spec,
        out_shape=jax.ShapeDtypeStruct((_N, _D), jnp.float32),
    )(x, W_in, b_in.reshape(1, _D))
    x0 = h

    agg = _sc_agg(h, src_p, dst_p)
    beta1 = math.log(_THETA / 1.0 + 1.0)
    h1 = pl.pallas_call(
        functools.partial(_mix_body, beta1),
        grid=grid,
        in_specs=[agg_spec, row_spec, w_spec],
        out_specs=row_spec,
        out_shape=jax.ShapeDtypeStruct((_N, _D), jnp.float32),
    )(agg, x0, W_conv1)

    agg2 = _sc_agg(h1, src_p, dst_p)
    beta2 = math.log(_THETA / 2.0 + 1.0)
    out = pl.pallas_call(
        functools.partial(_tail_body, beta2),
        grid=grid,
        in_specs=[agg_spec, row_spec, w_spec, wo_spec, bo_spec],
        out_specs=pl.BlockSpec((_ROW_BLK, _D_OUT), lambda i: (i, 0)),
        out_shape=jax.ShapeDtypeStruct((_N, _D_OUT), jnp.float32),
    )(agg2, x0, W_conv2, W_out, b_out.reshape(1, _D_OUT))
    return out


# speedup vs baseline: 1.2824x; 1.2824x over previous
import functools
import math

import jax
import jax.numpy as jnp
from jax import lax
from jax.experimental import pallas as pl
from jax.experimental.pallas import tpu as pltpu
from jax.experimental.pallas import tpu_sc as plsc

_N = 10000
_E = 320000
_D = 128
_D_OUT = 64
_ALPHA = 0.1
_THETA = 0.5

_NC = 2
_NS = 16
_NW = _NC * _NS
_G = 80
_NGRP = 128
_EPT = _G * _NGRP
_EPAD = _EPT * _NW
_NPAD = 10112
_RPT = _NPAD // _NS

_ROW_BLK = 1000
_NBUF = 4
_NIDX = 8
_LOOK = 4


def _sc_agg_body(h_hbm, src_hbm, dst_hbm, out_hbm,
                 isrc_v, idst_v, rows_v, agg_sh, gsem, ssem, isem):
    c = lax.axis_index("c")
    s = lax.axis_index("s")
    w = c * _NS + s

    def _zrow(i, _):
        for j in range(_D // 16):
            rows_v[0, i, pl.ds(j * 16, 16)] = jnp.zeros((16,), jnp.float32)
        return 0
    lax.fori_loop(0, _G, _zrow, 0)
    for k in range(_RPT // _G):
        pltpu.sync_copy(rows_v.at[0], agg_sh.at[pl.ds(s * _RPT + k * _G, _G)])
    _zrem = _RPT % _G
    if _zrem:
        pltpu.sync_copy(
            rows_v.at[0, pl.ds(0, _zrem)],
            agg_sh.at[pl.ds(s * _RPT + (_RPT // _G) * _G, _zrem)])
    plsc.subcore_barrier()

    def _fire_idx(g):
        slot = lax.rem(g, _NIDX)
        pltpu.async_copy(src_hbm.at[w, g], isrc_v.at[slot], isem)
        pltpu.async_copy(dst_hbm.at[w, g], idst_v.at[slot], isem)

    def _wait_idx(g):
        slot = lax.rem(g, _NIDX)
        pltpu.make_async_copy(src_hbm.at[w, g], isrc_v.at[slot], isem).wait()
        pltpu.make_async_copy(dst_hbm.at[w, g], idst_v.at[slot], isem).wait()

    def _fire_gather(g):
        pltpu.async_copy(h_hbm.at[isrc_v.at[lax.rem(g, _NIDX)]],
                         rows_v.at[lax.rem(g, _NBUF)], gsem)

    def _wait_gather(g):
        pltpu.make_async_copy(h_hbm.at[isrc_v.at[lax.rem(g, _NIDX)]],
                              rows_v.at[lax.rem(g, _NBUF)], gsem).wait()

    def _fire_scatter(g):
        pltpu.async_copy(rows_v.at[lax.rem(g, _NBUF)],
                         agg_sh.at[idst_v.at[lax.rem(g, _NIDX)]], ssem,
                         add=True)

    def _wait_scatter(g):
        pltpu.make_async_copy(rows_v.at[lax.rem(g, _NBUF)],
                              agg_sh.at[idst_v.at[lax.rem(g, _NIDX)]],
                              ssem).wait()

    for g in range(_LOOK + 3):
        _fire_idx(g)
    for g in range(3):
        _wait_idx(g)
        _fire_gather(g)
    _wait_gather(0)
    _fire_scatter(0)

    def _edge_grp(g, _):
        _wait_idx(g + 2)
        _fire_gather(g + 2)
        _fire_idx(g + _LOOK + 2)
        _wait_gather(g)
        _wait_scatter(g - 1)
        _fire_scatter(g)
        return 0
    lax.fori_loop(1, _NGRP - _LOOK - 2, _edge_grp, 0)

    for g in range(_NGRP - _LOOK - 2, _NGRP - 2):
        _wait_idx(g + 2)
        _fire_gather(g + 2)
        _wait_gather(g)
        _wait_scatter(g - 1)
        _fire_scatter(g)
    for g in range(_NGRP - 2, _NGRP):
        _wait_gather(g)
        _wait_scatter(g - 1)
        _fire_scatter(g)
    _wait_scatter(_NGRP - 1)
    plsc.subcore_barrier()

    pltpu.sync_copy(agg_sh.at[pl.ds(s * _RPT, _RPT)],
                    out_hbm.at[c, pl.ds(s * _RPT, _RPT)])


_sc_agg = pl.kernel(
    _sc_agg_body,
    out_type=jax.ShapeDtypeStruct((_NC, _NPAD, _D), jnp.float32),
    mesh=plsc.VectorSubcoreMesh(core_axis_name="c", subcore_axis_name="s"),
    scratch_types=[
        pltpu.VMEM((_NIDX, _G), jnp.int32),
        pltpu.VMEM((_NIDX, _G), jnp.int32),
        pltpu.VMEM((_NBUF, _G, _D), jnp.float32),
        pltpu.VMEM_SHARED((_NPAD, _D), jnp.float32),
        pltpu.SemaphoreType.DMA,
        pltpu.SemaphoreType.DMA,
        pltpu.SemaphoreType.DMA,
    ],
)


def _in_proj_body(x_ref, w_ref, b_ref, o_ref):
    o_ref[...] = jnp.maximum(
        jnp.dot(x_ref[...], w_ref[...], preferred_element_type=jnp.float32)
        + b_ref[...], 0.0)


def _mix_body(beta, agg_ref, x0_ref, w_ref, o_ref):
    m = (1.0 - _ALPHA) * (agg_ref[0] + agg_ref[1]) + _ALPHA * x0_ref[...]
    o_ref[...] = jnp.maximum(
        (1.0 - beta) * m
        + beta * jnp.dot(m, w_ref[...], preferred_element_type=jnp.float32),
        0.0)


def _tail_body(beta, agg_ref, x0_ref, w_ref, wo_ref, bo_ref, o_ref):
    m = (1.0 - _ALPHA) * (agg_ref[0] + agg_ref[1]) + _ALPHA * x0_ref[...]
    h2 = jnp.maximum(
        (1.0 - beta) * m
        + beta * jnp.dot(m, w_ref[...], preferred_element_type=jnp.float32),
        0.0)
    z = jnp.dot(h2, wo_ref[...], preferred_element_type=jnp.float32) + bo_ref[...]
    z = z - jnp.max(z, axis=-1, keepdims=True)
    o_ref[...] = z - jnp.log(jnp.sum(jnp.exp(z), axis=-1, keepdims=True))


def kernel(x, edge_index, W_in, b_in, W_out, b_out, W_conv1, W_conv2):
    src = edge_index[0].astype(jnp.int32)
    dst = edge_index[1].astype(jnp.int32)
    pad = _EPAD - _E
    pad_dst = _N + (jnp.arange(pad, dtype=jnp.int32) % (_NPAD - _N))
    pad_src = jnp.arange(pad, dtype=jnp.int32) % _N
    src_p = jnp.concatenate([src, pad_src]).reshape(_NW, _NGRP, _G)
    dst_p = jnp.concatenate([dst, pad_dst]).reshape(_NW, _NGRP, _G)

    w_spec = pl.BlockSpec((_D, _D), lambda i: (0, 0))
    b_spec = pl.BlockSpec((1, _D), lambda i: (0, 0))
    row_spec = pl.BlockSpec((_ROW_BLK, _D), lambda i: (i, 0))
    agg_spec = pl.BlockSpec((_NC, _ROW_BLK, _D), lambda i: (0, i, 0))
    wo_spec = pl.BlockSpec((_D, _D_OUT), lambda i: (0, 0))
    bo_spec = pl.BlockSpec((1, _D_OUT), lambda i: (0, 0))
    grid = (_N // _ROW_BLK,)

    h = pl.pallas_call(
        _in_proj_body,
        grid=grid,
        in_specs=[row_spec, w_spec, b_spec],
        out_specs=row_spec,
        out_shape=jax.ShapeDtypeStruct((_N, _D), jnp.float32),
    )(x, W_in, b_in.reshape(1, _D))
    x0 = h

    agg = _sc_agg(h, src_p, dst_p)
    beta1 = math.log(_THETA / 1.0 + 1.0)
    h1 = pl.pallas_call(
        functools.partial(_mix_body, beta1),
        grid=grid,
        in_specs=[agg_spec, row_spec, w_spec],
        out_specs=row_spec,
        out_shape=jax.ShapeDtypeStruct((_N, _D), jnp.float32),
    )(agg, x0, W_conv1)

    agg2 = _sc_agg(h1, src_p, dst_p)
    beta2 = math.log(_THETA / 2.0 + 1.0)
    out = pl.pallas_call(
        functools.partial(_tail_body, beta2),
        grid=grid,
        in_specs=[agg_spec, row_spec, w_spec, wo_spec, bo_spec],
        out_specs=pl.BlockSpec((_ROW_BLK, _D_OUT), lambda i: (i, 0)),
        out_shape=jax.ShapeDtypeStruct((_N, _D_OUT), jnp.float32),
    )(agg2, x0, W_conv2, W_out, b_out.reshape(1, _D_OUT))
    return out
